# baseline (speedup 1.0000x reference)
import jax
import jax.numpy as jnp
from jax import lax
from jax.experimental import pallas as pl
from jax.experimental.pallas import tpu as pltpu
from jax.experimental.pallas import tpu_sc as plsc

_NC = 2
_NS = 16
_NW = _NC * _NS

_CHUNK = 64
_NBUF = 3
_DEPTH = 2


def _make_gather(vocab, hidden, n_chunks):
    mesh = plsc.VectorSubcoreMesh(core_axis_name="c", subcore_axis_name="s")
    b_per_w = n_chunks * _CHUNK

    @pl.kernel(
        out_type=jax.ShapeDtypeStruct((_NW * b_per_w, hidden), jnp.float32),
        mesh=mesh,
        scratch_types=[
            pltpu.VMEM((n_chunks, _CHUNK), jnp.int32),
            pltpu.VMEM((_NBUF, _CHUNK, hidden), jnp.float32),
        ] + [pltpu.SemaphoreType.DMA] * (2 * _NBUF),
    )
    def gather(idx_hbm, table_hbm, out_hbm, idx_v, rows_v, *sems):
        wid = lax.axis_index("s") * _NC + lax.axis_index("c")
        pltpu.sync_copy(idx_hbm.at[wid], idx_v)
        base = wid * b_per_w
        gsem = sems[:_NBUF]
        wsem = sems[_NBUF:]

        gathers = [None] * n_chunks
        writes = [None] * n_chunks
        for c in range(min(_DEPTH, n_chunks)):
            gathers[c] = pltpu.async_copy(
                table_hbm.at[idx_v.at[c]], rows_v.at[c % _NBUF],
                gsem[c % _NBUF])
        for c in range(n_chunks):
            b = c % _NBUF
            gathers[c].wait()
            writes[c] = pltpu.async_copy(
                rows_v.at[b], out_hbm.at[pl.ds(base + c * _CHUNK, _CHUNK)],
                wsem[b])
            if c + _DEPTH < n_chunks:
                prev = c + _DEPTH - _NBUF
                if prev >= 0:
                    writes[prev].wait()
                gathers[c + _DEPTH] = pltpu.async_copy(
                    table_hbm.at[idx_v.at[c + _DEPTH]],
                    rows_v.at[(c + _DEPTH) % _NBUF],
                    gsem[(c + _DEPTH) % _NBUF])
        for c in range(n_chunks):
            if writes[c] is not None and c >= n_chunks - _NBUF:
                writes[c].wait()

    return gather


def kernel(input_ids, table):
    batch, seq = input_ids.shape
    vocab, hidden = table.shape
    total = batch * seq
    assert total % (_NW * _CHUNK) == 0
    n_chunks = total // (_NW * _CHUNK)
    idx3 = input_ids.reshape(_NW, n_chunks, _CHUNK).astype(jnp.int32)
    out = _make_gather(vocab, hidden, n_chunks)(idx3, table)
    return out.reshape(batch, seq, hidden)


# speedup vs baseline: 1.0137x; 1.0137x over previous
import jax
import jax.numpy as jnp
from jax import lax
from jax.experimental import pallas as pl
from jax.experimental.pallas import tpu as pltpu
from jax.experimental.pallas import tpu_sc as plsc

_NC = 2
_NS = 16
_NW = _NC * _NS

_CHUNK = 32
_NBUF = 7
_DEPTH = 6


def _make_gather(vocab, hidden, n_chunks):
    mesh = plsc.VectorSubcoreMesh(core_axis_name="c", subcore_axis_name="s")
    b_per_w = n_chunks * _CHUNK

    @pl.kernel(
        out_type=jax.ShapeDtypeStruct((_NW * b_per_w, hidden), jnp.float32),
        mesh=mesh,
        scratch_types=[
            pltpu.VMEM((n_chunks, _CHUNK), jnp.int32),
            pltpu.VMEM((_NBUF, _CHUNK, hidden), jnp.float32),
        ] + [pltpu.SemaphoreType.DMA] * (2 * _NBUF),
    )
    def gather(idx_hbm, table_hbm, out_hbm, idx_v, rows_v, *sems):
        wid = lax.axis_index("s") * _NC + lax.axis_index("c")
        pltpu.sync_copy(idx_hbm.at[wid], idx_v)
        base = wid * b_per_w
        gsem = sems[:_NBUF]
        wsem = sems[_NBUF:]

        gathers = [None] * n_chunks
        writes = [None] * n_chunks
        for c in range(min(_DEPTH, n_chunks)):
            gathers[c] = pltpu.async_copy(
                table_hbm.at[idx_v.at[c]], rows_v.at[c % _NBUF],
                gsem[c % _NBUF])
        for c in range(n_chunks):
            b = c % _NBUF
            gathers[c].wait()
            writes[c] = pltpu.async_copy(
                rows_v.at[b], out_hbm.at[pl.ds(base + c * _CHUNK, _CHUNK)],
                wsem[b])
            if c + _DEPTH < n_chunks:
                prev = c + _DEPTH - _NBUF
                if prev >= 0:
                    writes[prev].wait()
                gathers[c + _DEPTH] = pltpu.async_copy(
                    table_hbm.at[idx_v.at[c + _DEPTH]],
                    rows_v.at[(c + _DEPTH) % _NBUF],
                    gsem[(c + _DEPTH) % _NBUF])
        for c in range(n_chunks):
            if writes[c] is not None and c >= n_chunks - _NBUF:
                writes[c].wait()

    return gather


def kernel(input_ids, table):
    batch, seq = input_ids.shape
    vocab, hidden = table.shape
    total = batch * seq
    assert total % (_NW * _CHUNK) == 0
    n_chunks = total // (_NW * _CHUNK)
    idx3 = input_ids.reshape(_NW, n_chunks, _CHUNK).astype(jnp.int32)
    out = _make_gather(vocab, hidden, n_chunks)(idx3, table)
    return out.reshape(batch, seq, hidden)
